# Initial kernel scaffold; baseline (speedup 1.0000x reference)
import jax
import jax.numpy as jnp
from jax.experimental import pallas as pl


def kernel(x, A, W1, b1, W2, b2):
    raise NotImplementedError("write your pallas kernel here")


# speedup vs baseline: 1.4738x; 1.4738x over previous
import jax
import jax.numpy as jnp
from jax.experimental import pallas as pl


def _fold_kernel(m1a_ref, m2_ref, out_ref):
    out_ref[...] = jnp.dot(
        m1a_ref[...], m2_ref[...], preferred_element_type=jnp.float32
    )


def _gemm_kernel(x_ref, m_ref, b_ref, out_ref):
    out_ref[...] = (
        jnp.dot(x_ref[...], m_ref[...], preferred_element_type=jnp.float32)
        + b_ref[...]
    )


ROWS_PER_BLOCK = 1024


def kernel(x, A, W1, b1, W2, b2):
    n, t, v, ci = x.shape
    k = A.shape[0]
    h = W1.shape[0] // k
    co = W2.shape[0] // k

    W1r = W1.reshape(k, h, ci).transpose(0, 2, 1)
    W2r = W2.reshape(k, co, h).transpose(0, 2, 1)
    M1 = (A[:, :, None, :, None] * W1r[:, None, :, None, :]).sum(0)
    M1 = M1.reshape(v * ci, v * h)
    M2 = (A[:, :, None, :, None] * W2r[:, None, :, None, :]).sum(0)
    M2 = M2.reshape(v * h, v * co)
    S = A.sum(axis=1)
    b1r = b1.reshape(k, h)
    b2r = b2.reshape(k, co)
    B1 = (S.T[:, :, None] * b1r[None, :, :]).sum(1).reshape(1, v * h)
    b2eff = (S.T[:, :, None] * b2r[None, :, :]).sum(1).reshape(1, v * co)

    m1a = jnp.concatenate([M1, B1], axis=0)

    mfold = pl.pallas_call(
        _fold_kernel,
        out_shape=jax.ShapeDtypeStruct((v * ci + 1, v * co), jnp.float32),
    )(m1a, M2)

    M = mfold[: v * ci]
    beff = mfold[v * ci:] + b2eff

    rows = n * t
    X = x.reshape(rows, v * ci)
    r = ROWS_PER_BLOCK
    out = pl.pallas_call(
        _gemm_kernel,
        grid=(rows // r,),
        in_specs=[
            pl.BlockSpec((r, v * ci), lambda i: (i, 0)),
            pl.BlockSpec((v * ci, v * co), lambda i: (0, 0)),
            pl.BlockSpec((1, v * co), lambda i: (0, 0)),
        ],
        out_specs=pl.BlockSpec((r, v * co), lambda i: (i, 0)),
        out_shape=jax.ShapeDtypeStruct((rows, v * co), jnp.float32),
    )(X, M, beff)

    return out.reshape(n, t, v * co)


# speedup vs baseline: 2.2299x; 1.5130x over previous
import jax
import jax.numpy as jnp
from jax.experimental import pallas as pl


def _fold_kernel(m1a_ref, m2_ref, out_ref):
    out_ref[...] = jnp.dot(
        m1a_ref[...], m2_ref[...], preferred_element_type=jnp.float32
    )


def _gemm_kernel(x_ref, m_ref, b_ref, out_ref):
    for j in range(x_ref.shape[0]):
        out_ref[j] = (
            jnp.dot(x_ref[j], m_ref[...], preferred_element_type=jnp.float32)
            + b_ref[...]
        )


SAMPLES_PER_BLOCK = 8


def kernel(x, A, W1, b1, W2, b2):
    n, t, v, ci = x.shape
    k = A.shape[0]
    h = W1.shape[0] // k
    co = W2.shape[0] // k

    W1r = W1.reshape(k, h, ci).transpose(0, 2, 1)
    W2r = W2.reshape(k, co, h).transpose(0, 2, 1)
    M1 = (A[:, :, None, :, None] * W1r[:, None, :, None, :]).sum(0)
    M1 = M1.reshape(v * ci, v * h)
    M2 = (A[:, :, None, :, None] * W2r[:, None, :, None, :]).sum(0)
    M2 = M2.reshape(v * h, v * co)
    S = A.sum(axis=1)
    b1r = b1.reshape(k, h)
    b2r = b2.reshape(k, co)
    B1 = (S.T[:, :, None] * b1r[None, :, :]).sum(1).reshape(1, v * h)
    b2eff = (S.T[:, :, None] * b2r[None, :, :]).sum(1).reshape(1, v * co)

    m1a = jnp.concatenate([M1, B1], axis=0)

    mfold = pl.pallas_call(
        _fold_kernel,
        out_shape=jax.ShapeDtypeStruct((v * ci + 1, v * co), jnp.float32),
    )(m1a, M2)

    M = mfold[: v * ci]
    beff = mfold[v * ci:] + b2eff

    X = x.reshape(n, t, v * ci)
    bn = SAMPLES_PER_BLOCK
    out = pl.pallas_call(
        _gemm_kernel,
        grid=(n // bn,),
        in_specs=[
            pl.BlockSpec((bn, t, v * ci), lambda i: (i, 0, 0)),
            pl.BlockSpec((v * ci, v * co), lambda i: (0, 0)),
            pl.BlockSpec((1, v * co), lambda i: (0, 0)),
        ],
        out_specs=pl.BlockSpec((bn, t, v * co), lambda i: (i, 0, 0)),
        out_shape=jax.ShapeDtypeStruct((n, t, v * co), jnp.float32),
    )(X, M, beff)

    return out
